# baseline (speedup 1.0000x reference)
import functools

import jax
import jax.numpy as jnp
from jax import lax
from jax.experimental import pallas as pl
from jax.experimental.pallas import tpu as pltpu
from jax.experimental.pallas import tpu_sc as plsc

D = 64
ROW = 128
GROUP = 4


def _make_gather(n_rows: int):
    info = plsc.get_sparse_core_info()
    nw = info.num_cores * info.num_subcores
    rows_per_w = n_rows // nw
    n_groups = rows_per_w // GROUP
    mesh = plsc.VectorSubcoreMesh(core_axis_name="c", subcore_axis_name="s")

    @functools.partial(
        pl.kernel,
        mesh=mesh,
        out_type=jax.ShapeDtypeStruct((n_rows * ROW, D), jnp.float32),
        scratch_types=[
            pltpu.VMEM((GROUP, ROW), jnp.int32),
            pltpu.VMEM((GROUP * ROW, D), jnp.float32),
            pltpu.SemaphoreType.DMA,
        ],
        compiler_params=pltpu.CompilerParams(use_tc_tiling_on_sc=False),
    )
    def k(table_hbm, idx_hbm, out_hbm, idx_v, rows_v, sem):
        wid = lax.axis_index("s") * info.num_cores + lax.axis_index("c")
        start = wid * rows_per_w

        def body(g, _):
            r = start + g * GROUP
            pltpu.sync_copy(idx_hbm.at[pl.ds(r, GROUP)], idx_v)
            for j in range(GROUP):
                pltpu.async_copy(
                    table_hbm.at[idx_v.at[j]],
                    rows_v.at[pl.ds(j * ROW, ROW)],
                    sem,
                ).wait()
            pltpu.sync_copy(rows_v, out_hbm.at[pl.ds(r * ROW, GROUP * ROW)])
            return ()

        lax.fori_loop(0, n_groups, body, (), unroll=False)

    return k


def kernel(token_tensor, table):
    b, l = token_tensor.shape
    idx = token_tensor.reshape(-1, ROW)
    out = _make_gather(idx.shape[0])(table, idx)
    return out.reshape(b, l, D)


# speedup vs baseline: 1.1176x; 1.1176x over previous
import functools

import jax
import jax.numpy as jnp
from jax import lax
from jax.experimental import pallas as pl
from jax.experimental.pallas import tpu as pltpu
from jax.experimental.pallas import tpu_sc as plsc

D = 64
ROW = 128
GROUP = 4
C = GROUP * ROW


def _make_gather(n_rows: int):
    info = plsc.get_sparse_core_info()
    nw = info.num_cores * info.num_subcores
    rows_per_w = n_rows // nw
    n_steps = rows_per_w // GROUP
    n_pairs = n_steps // 2
    mesh = plsc.VectorSubcoreMesh(core_axis_name="c", subcore_axis_name="s")

    @functools.partial(
        pl.kernel,
        mesh=mesh,
        out_type=jax.ShapeDtypeStruct((n_rows * ROW, D), jnp.float32),
        scratch_types=[
            pltpu.VMEM((GROUP, ROW), jnp.int32),
            pltpu.VMEM((GROUP, ROW), jnp.int32),
            pltpu.VMEM((C, D), jnp.float32),
            pltpu.VMEM((C, D), jnp.float32),
            pltpu.SemaphoreType.DMA,
            pltpu.SemaphoreType.DMA,
            pltpu.SemaphoreType.DMA,
            pltpu.SemaphoreType.DMA,
            pltpu.SemaphoreType.DMA,
            pltpu.SemaphoreType.DMA,
        ],
        compiler_params=pltpu.CompilerParams(use_tc_tiling_on_sc=False),
    )
    def k(table_hbm, idx_hbm, out_hbm, idx0, idx1, rows0, rows1,
          si0, si1, sg0, sg1, so0, so1):
        wid = lax.axis_index("s") * info.num_cores + lax.axis_index("c")
        start = wid * rows_per_w

        def load_idx(r, idx_v, si):
            pltpu.async_copy(idx_hbm.at[pl.ds(r, GROUP)], idx_v, si)

        def half(r, idx_v, rows_v, si, sg, so, first, last):
            pltpu.make_async_copy(idx_hbm.at[pl.ds(r, GROUP)], idx_v, si).wait()
            if not first:
                pltpu.make_async_copy(
                    rows_v, out_hbm.at[pl.ds(r * ROW, C)], so).wait()
            cps = [
                pltpu.async_copy(
                    table_hbm.at[idx_v.at[j]],
                    rows_v.at[pl.ds(j * ROW, ROW)],
                    sg,
                )
                for j in range(GROUP)
            ]
            for cp in cps:
                cp.wait()
            if not last:
                load_idx(r + 2 * GROUP, idx_v, si)
            pltpu.async_copy(rows_v, out_hbm.at[pl.ds(r * ROW, C)], so)

        load_idx(start, idx0, si0)
        load_idx(start + GROUP, idx1, si1)

        half(start, idx0, rows0, si0, sg0, so0, first=True, last=False)
        half(start + GROUP, idx1, rows1, si1, sg1, so1, first=True, last=False)

        def body(p, _):
            r = start + 2 * p * GROUP
            half(r, idx0, rows0, si0, sg0, so0, first=False, last=False)
            half(r + GROUP, idx1, rows1, si1, sg1, so1, first=False, last=False)
            return ()

        lax.fori_loop(1, n_pairs - 1, body, (), unroll=False)

        r = start + 2 * (n_pairs - 1) * GROUP
        half(r, idx0, rows0, si0, sg0, so0, first=False, last=True)
        half(r + GROUP, idx1, rows1, si1, sg1, so1, first=False, last=True)

        pltpu.make_async_copy(rows0, out_hbm.at[pl.ds(0, C)], so0).wait()
        pltpu.make_async_copy(rows1, out_hbm.at[pl.ds(0, C)], so1).wait()

    return k


def kernel(token_tensor, table):
    b, l = token_tensor.shape
    idx = token_tensor.reshape(-1, ROW)
    out = _make_gather(idx.shape[0])(table, idx)
    return out.reshape(b, l, D)


# speedup vs baseline: 1.4924x; 1.3354x over previous
import functools

import jax
import jax.numpy as jnp
from jax import lax
from jax.experimental import pallas as pl
from jax.experimental.pallas import tpu as pltpu
from jax.experimental.pallas import tpu_sc as plsc

D = 64
ROW = 128
GROUP = 4
C = GROUP * ROW


def _make_gather(n_rows: int):
    info = plsc.get_sparse_core_info()
    nw = info.num_cores * info.num_subcores
    rows_per_w = n_rows // nw
    n_steps = rows_per_w // GROUP
    n_pairs = n_steps // 2
    mesh = plsc.VectorSubcoreMesh(core_axis_name="c", subcore_axis_name="s")

    @functools.partial(
        pl.kernel,
        mesh=mesh,
        out_type=jax.ShapeDtypeStruct((n_rows * ROW, 2 * D), jnp.float32),
        scratch_types=[
            pltpu.VMEM((GROUP, ROW), jnp.int32),
            pltpu.VMEM((GROUP, ROW), jnp.int32),
            pltpu.VMEM((C, D), jnp.float32),
            pltpu.VMEM((C, D), jnp.float32),
            pltpu.SemaphoreType.DMA,
            pltpu.SemaphoreType.DMA,
            pltpu.SemaphoreType.DMA,
            pltpu.SemaphoreType.DMA,
            pltpu.SemaphoreType.DMA,
            pltpu.SemaphoreType.DMA,
        ],
        compiler_params=pltpu.CompilerParams(use_tc_tiling_on_sc=False),
    )
    def k(table_hbm, idx_hbm, out_hbm, idx0, idx1, rows0, rows1,
          si0, si1, sg0, sg1, so0, so1):
        wid = lax.axis_index("s") * info.num_cores + lax.axis_index("c")
        start = wid * rows_per_w

        def load_idx(r, idx_v, si):
            pltpu.async_copy(idx_hbm.at[pl.ds(r, GROUP)], idx_v, si)

        def half(r, idx_v, rows_v, si, sg, so, first, last):
            dst = out_hbm.at[pl.ds(r * ROW, C), pl.ds(0, D)]
            pltpu.make_async_copy(idx_hbm.at[pl.ds(r, GROUP)], idx_v, si).wait()
            if not first:
                pltpu.make_async_copy(rows_v, dst, so).wait()
            cps = [
                pltpu.async_copy(
                    table_hbm.at[idx_v.at[j]],
                    rows_v.at[pl.ds(j * ROW, ROW)],
                    sg,
                )
                for j in range(GROUP)
            ]
            for cp in cps:
                cp.wait()
            if not last:
                load_idx(r + 2 * GROUP, idx_v, si)
            pltpu.async_copy(rows_v, dst, so)

        load_idx(start, idx0, si0)
        load_idx(start + GROUP, idx1, si1)

        half(start, idx0, rows0, si0, sg0, so0, first=True, last=False)
        half(start + GROUP, idx1, rows1, si1, sg1, so1, first=True, last=False)

        def body(p, _):
            r = start + 2 * p * GROUP
            half(r, idx0, rows0, si0, sg0, so0, first=False, last=False)
            half(r + GROUP, idx1, rows1, si1, sg1, so1, first=False, last=False)
            return ()

        lax.fori_loop(1, n_pairs - 1, body, (), unroll=False)

        r = start + 2 * (n_pairs - 1) * GROUP
        half(r, idx0, rows0, si0, sg0, so0, first=False, last=True)
        half(r + GROUP, idx1, rows1, si1, sg1, so1, first=False, last=True)

        pltpu.make_async_copy(
            rows0, out_hbm.at[pl.ds(0, C), pl.ds(0, D)], so0).wait()
        pltpu.make_async_copy(
            rows1, out_hbm.at[pl.ds(0, C), pl.ds(0, D)], so1).wait()

    return k


def kernel(token_tensor, table):
    b, l = token_tensor.shape
    idx = token_tensor.reshape(-1, ROW)
    out2 = _make_gather(idx.shape[0])(table, idx)
    return out2[:, :D].reshape(b, l, D)
